# Initial kernel scaffold; baseline (speedup 1.0000x reference)
import jax
import jax.numpy as jnp
from jax.experimental import pallas as pl


def kernel(atom_feats, mass, bond_feats, pos, edge_index, node2graph, atom_tables, mass_centers, mass_W, bond_tables, dist_centers, dist_W, Wq, Wk, Wv, Wo, Wb, We):
    raise NotImplementedError("write your pallas kernel here")


# speedup vs baseline: 2.1824x; 2.1824x over previous
import jax
import jax.numpy as jnp
from jax.experimental import pallas as pl
from jax.experimental.pallas import tpu as pltpu

_B = 16
_S = 48
_N = _B * _S
_E = 3072
_H = 128
_NH = 8
_DH = _H // _NH
_L = 6
_K = 16
_EG = _E // _B
_F32 = jnp.float32


def _ln(x):
    m = x.mean(-1, keepdims=True)
    v = ((x - m) ** 2).mean(-1, keepdims=True)
    return (x - m) / jnp.sqrt(v + 1e-5)


def _body(af_ref, mass_ref, pos_ref, bf_ref, pidx_ref,
          atomtab_ref, massc_ref, massW_ref, bondtab_ref, distc_ref, distW_ref,
          wq_ref, wk_ref, wv_ref, wo_ref, wb_ref, we_ref, out_ref):
    af = af_ref[0]
    x = jnp.zeros((_S, _H), dtype=_F32)
    iota_n = jax.lax.broadcasted_iota(jnp.int32, (_S, 32), 1)
    for f in range(9):
        ohf = (af[:, f:f + 1] == iota_n).astype(_F32)
        x = x + jnp.dot(ohf, atomtab_ref[f], preferred_element_type=_F32)
    mass = mass_ref[0]
    rbf_m = jnp.exp(-10.0 * (mass - massc_ref[:]) ** 2)
    x = x + jnp.dot(rbf_m, massW_ref[:], preferred_element_type=_F32)

    pos = pos_ref[0]
    p_i = jnp.broadcast_to(pos[:, None, :], (_S, _S, 3)).reshape(_S * _S, 3)
    p_j = jnp.broadcast_to(pos[None, :, :], (_S, _S, 3)).reshape(_S * _S, 3)
    d2 = ((p_i - p_j) ** 2).sum(axis=-1, keepdims=True)
    dist = jnp.sqrt(d2 + 1e-9)
    rbf_d = jnp.exp(-10.0 * (dist - distc_ref[:]) ** 2)
    e = jnp.dot(rbf_d, distW_ref[:], preferred_element_type=_F32)

    bf = bf_ref[0]
    iota_b = jax.lax.broadcasted_iota(jnp.int32, (_EG, 8), 1)
    e_emb = jnp.zeros((_EG, _H), dtype=_F32)
    for f in range(3):
        ohf = (bf[:, f:f + 1] == iota_b).astype(_F32)
        e_emb = e_emb + jnp.dot(ohf, bondtab_ref[f], preferred_element_type=_F32)
    pidx = pidx_ref[0]
    iota_p = jax.lax.broadcasted_iota(jnp.int32, (_EG, _S * _S), 1)
    ohs = (pidx == iota_p).astype(_F32)
    e = e + jax.lax.dot_general(ohs, e_emb, (((0,), (0,)), ((), ())),
                                preferred_element_type=_F32)

    sel = (jax.lax.broadcasted_iota(jnp.int32, (_H, _NH), 0) // _DH
           == jax.lax.broadcasted_iota(jnp.int32, (_H, _NH), 1)).astype(_F32)
    sel_t = (jax.lax.broadcasted_iota(jnp.int32, (_NH, _H), 1) // _DH
             == jax.lax.broadcasted_iota(jnp.int32, (_NH, _H), 0)).astype(_F32)

    scale = 1.0 / (float(_DH) ** 0.5)
    for l in range(_L):
        xn = _ln(x)
        q = jnp.dot(xn, wq_ref[l], preferred_element_type=_F32)
        k = jnp.dot(xn, wk_ref[l], preferred_element_type=_F32)
        v = jnp.dot(xn, wv_ref[l], preferred_element_type=_F32)
        q2 = jnp.broadcast_to(q[:, None, :], (_S, _S, _H)).reshape(_S * _S, _H)
        k2 = jnp.broadcast_to(k[None, :, :], (_S, _S, _H)).reshape(_S * _S, _H)
        qk = jnp.dot(q2 * k2, sel, preferred_element_type=_F32)
        eb = jnp.dot(e, wb_ref[l], preferred_element_type=_F32)
        lg = qk * scale + eb
        t3 = lg.reshape(_S, _S, _NH)
        t3 = t3 - jnp.max(t3, axis=1, keepdims=True)
        a3 = jnp.exp(t3)
        a3 = a3 / jnp.sum(a3, axis=1, keepdims=True)
        a_flat = a3.reshape(_S * _S, _NH)
        a2 = jnp.dot(a_flat, sel_t, preferred_element_type=_F32)
        v2 = jnp.broadcast_to(v[None, :, :], (_S, _S, _H)).reshape(_S * _S, _H)
        out = (a2 * v2).reshape(_S, _S, _H).sum(axis=1)
        x = x + jnp.dot(out, wo_ref[l], preferred_element_type=_F32)
        m = x[:, None, :] + x[None, :, :] + e.reshape(_S, _S, _H)
        relu_m = jnp.maximum(m, 0.0).reshape(_S * _S, _H)
        e = e + jnp.dot(relu_m, we_ref[l], preferred_element_type=_F32)

    out_ref[0] = x


def kernel(atom_feats, mass, bond_feats, pos, edge_index, node2graph, atom_tables,
           mass_centers, mass_W, bond_tables, dist_centers, dist_W, Wq, Wk, Wv, Wo, Wb, We):
    af = atom_feats.astype(jnp.int32).reshape(_B, _S, 9)
    mass3 = mass.reshape(_B, _S, 1)
    bf = bond_feats.astype(jnp.int32).reshape(_B, _EG, 3)
    pos3 = pos.reshape(_B, _S, 3)
    src = edge_index[0].astype(jnp.int32)
    dst = edge_index[1].astype(jnp.int32)
    pidx = ((src % _S) * _S + (dst % _S)).reshape(_B, _EG, 1)
    massc = mass_centers.reshape(1, _K).astype(_F32)
    distc = dist_centers.reshape(1, _K).astype(_F32)

    grid = (_B,)
    full = lambda shape: pl.BlockSpec(shape, lambda b: (0,) * len(shape))
    perg = lambda shape: pl.BlockSpec((1,) + shape, lambda b: (b,) + (0,) * len(shape))

    return pl.pallas_call(
        _body,
        grid=grid,
        in_specs=[
            perg((_S, 9)),
            perg((_S, 1)),
            perg((_S, 3)),
            perg((_EG, 3)),
            perg((_EG, 1)),
            full((9, 32, _H)),
            full((1, _K)),
            full((_K, _H)),
            full((3, 8, _H)),
            full((1, _K)),
            full((_K, _H)),
            full((_L, _H, _H)),
            full((_L, _H, _H)),
            full((_L, _H, _H)),
            full((_L, _H, _H)),
            full((_L, _H, _NH)),
            full((_L, _H, _H)),
        ],
        out_specs=pl.BlockSpec((1, _S, _H), lambda b: (b, 0, 0)),
        out_shape=jax.ShapeDtypeStruct((_B, _S, _H), _F32),
        compiler_params=pltpu.CompilerParams(dimension_semantics=("parallel",)),
    )(af, mass3, pos3, bf, pidx, atom_tables, massc, mass_W, bond_tables,
      distc, dist_W, Wq, Wk, Wv, Wo, Wb, We)
